# baseline (speedup 1.0000x reference)
import functools

import jax
import jax.numpy as jnp
from jax import lax
from jax.experimental import pallas as pl
from jax.experimental.pallas import tpu as pltpu
from jax.experimental.pallas import tpu_sc as plsc

B = 16384
F = 26
V = 100000
D = 64
NC, NS, L = 2, 16, 16
NW = NC * NS
SPW = B // NW
C = 16
NCH = SPW // C

VCOLS = V // 128
VTAIL = V - VCOLS * 128
KMAX = VCOLS // NW + 1

_mesh = plsc.VectorSubcoreMesh(core_axis_name="c", subcore_axis_name="s")
_params = pltpu.CompilerParams(use_tc_tiling_on_sc=True, needs_layout_passes=False)


@functools.partial(
    pl.kernel,
    mesh=_mesh,
    out_type=jax.ShapeDtypeStruct((F, V // 2, 2 * D), jnp.float32),
    scratch_types=[
        pltpu.VMEM((D, 128), jnp.float32),
        pltpu.VMEM((D, 128), jnp.float32),
        pltpu.VMEM((D, 128), jnp.float32),
        pltpu.VMEM((D, 128), jnp.float32),
        pltpu.VMEM((D, VTAIL), jnp.float32),
        pltpu.VMEM((VTAIL // 2, 128), jnp.float32),
        pltpu.SemaphoreType.DMA,
        pltpu.SemaphoreType.DMA,
    ],
    compiler_params=_params,
)
def _convert(
    tabt_hbm, tab2_hbm, in_v0, in_v1, out_v0, out_v1, tin_v, tout_v, sem_i, sem_o
):
    wid = lax.axis_index("s") * NC + lax.axis_index("c")
    lanes = lax.iota(jnp.int32, L)
    dbases = [lanes + ((16 * k) % D) for k in range(2 * D // L)]
    in_bufs, out_bufs = [in_v0, in_v1], [out_v0, out_v1]

    def load(f, col, buf):
        off = pl.multiple_of(col * 128, 128)
        return pltpu.async_copy(
            tabt_hbm.at[f, :, pl.ds(off, 128)], in_bufs[buf], sem_i
        )

    def transpose(in_ref, out_ref, nrow):
        @pl.loop(0, nrow)
        def _row(jl):
            vlo = jnp.full((L,), 0, jnp.int32) + (2 * jl)
            vhi = vlo + 1
            for k in range(2 * D // L):
                vv = vlo if k < D // L else vhi
                out_ref[jl, pl.ds(k * L, L)] = plsc.load_gather(
                    in_ref, [dbases[k], vv]
                )

    def store(f, col, buf):
        off = pl.multiple_of(col * D, D)
        return pltpu.async_copy(
            out_bufs[buf], tab2_hbm.at[f, pl.ds(off, D), :], sem_o
        )

    @pl.loop(0, KMAX)
    def _k(k):
        col = wid + k * NW

        @pl.when(col < VCOLS)
        def _():
            h_in = load(0, col, 0)
            h_outs = [None, None]
            for f in range(F):
                b = f % 2
                if f + 1 < F:
                    h_next = load(f + 1, col, 1 - b)
                h_in.wait()
                if h_outs[b] is not None:
                    h_outs[b].wait()
                transpose(in_bufs[b], out_bufs[b], D)
                h_outs[b] = store(f, col, b)
                if f + 1 < F:
                    h_in = h_next
            for h in h_outs:
                if h is not None:
                    h.wait()

    @pl.when(wid == 0)
    def _tail():
        for f in range(F):
            pltpu.async_copy(
                tabt_hbm.at[f, :, pl.ds(VCOLS * 128, VTAIL)], tin_v, sem_i
            ).wait()
            transpose(tin_v, tout_v, VTAIL // 2)
            pltpu.async_copy(
                tout_v, tab2_hbm.at[f, pl.ds(VCOLS * D, VTAIL // 2), :], sem_o
            ).wait()


@functools.partial(
    pl.kernel,
    mesh=_mesh,
    out_type=jax.ShapeDtypeStruct((B, D), jnp.float32),
    scratch_types=[
        pltpu.VMEM((F * C,), jnp.int32),
        pltpu.VMEM((F, C), jnp.int32),
        pltpu.VMEM((F * C, 2 * D), jnp.float32),
        pltpu.VMEM((C, D), jnp.float32),
        pltpu.SemaphoreType.DMA,
    ],
    compiler_params=_params,
)
def _emb(x_hbm, tab_hbm, out_hbm, idx_v, idx2_v, rows_v, out_v, sem):
    wid = lax.axis_index("s") * NC + lax.axis_index("c")

    @pl.loop(0, NCH)
    def _chunk(c):
        pltpu.sync_copy(x_hbm.at[wid, c], idx_v)
        for f in range(F):
            for k in range(C // L):
                sl = pl.ds(k * L, L)
                idx2_v[f, sl] = lax.shift_right_logical(
                    idx_v[pl.ds(f * C + k * L, L)], 1
                )
        copies = [
            pltpu.async_copy(
                tab_hbm.at[f].at[idx2_v.at[f]],
                rows_v.at[pl.ds(f * C, C), :],
                sem,
            )
            for f in range(F)
        ]
        for cp in copies:
            cp.wait()

        lanes = lax.iota(jnp.int32, L)

        @pl.loop(0, C)
        def _acc(s):
            svec = jnp.full((L,), 0, jnp.int32) + s
            accs = None
            for f in range(F):
                rsplat = svec + (f * C)
                raw = plsc.load_gather(idx_v, [rsplat])
                off = lax.shift_left((raw & 1), 6)
                vals = [
                    plsc.load_gather(rows_v, [rsplat, off + (v * L) + lanes])
                    for v in range(D // L)
                ]
                accs = vals if accs is None else [a + b for a, b in zip(accs, vals)]
            for v in range(D // L):
                out_v[s, pl.ds(v * L, L)] = accs[v]

        pltpu.sync_copy(out_v, out_hbm.at[pl.ds(wid * SPW + c * C, C), :])


def kernel(x, tables):
    xt = (
        x.astype(jnp.int32)
        .T.reshape(F, NW, NCH, C)
        .transpose(1, 2, 0, 3)
        .reshape(NW, NCH, F * C)
    )
    tabt = jnp.swapaxes(tables, 1, 2)
    tab2 = _convert(tabt)
    return _emb(xt, tab2)


# speedup vs baseline: 2.9122x; 2.9122x over previous
import functools

import jax
import jax.numpy as jnp
from jax import lax
from jax.experimental import pallas as pl
from jax.experimental.pallas import tpu as pltpu
from jax.experimental.pallas import tpu_sc as plsc

B = 16384
F = 26
V = 100000
D = 64
NC, NS, L = 2, 16, 16
NW = NC * NS
SPW = B // NW
C = 16
NCH = SPW // C

_mesh = plsc.VectorSubcoreMesh(core_axis_name="c", subcore_axis_name="s")


@functools.partial(
    pl.kernel,
    mesh=_mesh,
    out_type=jax.ShapeDtypeStruct((B, D), jnp.float32),
    scratch_types=[
        pltpu.VMEM((F * C,), jnp.int32),
        pltpu.VMEM((F, C), jnp.int32),
        pltpu.VMEM((F * C, 2 * D), jnp.float32),
        pltpu.VMEM((C, D), jnp.float32),
        pltpu.SemaphoreType.DMA,
    ],
    compiler_params=pltpu.CompilerParams(
        use_tc_tiling_on_sc=True, needs_layout_passes=False
    ),
)
def _emb(x_hbm, tab_hbm, out_hbm, idx_v, idx2_v, rows_v, out_v, sem):
    wid = lax.axis_index("s") * NC + lax.axis_index("c")

    @pl.loop(0, NCH)
    def _chunk(c):
        pltpu.sync_copy(x_hbm.at[wid, c], idx_v)
        for f in range(F):
            for k in range(C // L):
                idx2_v[f, pl.ds(k * L, L)] = idx_v[pl.ds(f * C + k * L, L)] + (f * V)
        copies = [
            pltpu.async_copy(
                tab_hbm.at[idx2_v.at[f]],
                rows_v.at[pl.ds(f * C, C), :],
                sem,
            )
            for f in range(F)
        ]
        for cp in copies:
            cp.wait()

        @pl.loop(0, C)
        def _acc(s):
            accs = None
            for f in range(F):
                row = f * C + s
                vals = [rows_v[row, pl.ds(v * L, L)] for v in range(D // L)]
                accs = vals if accs is None else [a + b for a, b in zip(accs, vals)]
            for v in range(D // L):
                out_v[s, pl.ds(v * L, L)] = accs[v]

        pltpu.sync_copy(out_v, out_hbm.at[pl.ds(wid * SPW + c * C, C), :])


def kernel(x, tables):
    xt = (
        x.astype(jnp.int32)
        .T.reshape(F, NW, NCH, C)
        .transpose(1, 2, 0, 3)
        .reshape(NW, NCH, F * C)
    )
    tabp = jnp.pad(tables, ((0, 0), (0, 0), (0, D))).reshape(F * V, 2 * D)
    return _emb(xt, tabp)


# speedup vs baseline: 3.0543x; 1.0488x over previous
import functools

import jax
import jax.numpy as jnp
from jax import lax
from jax.experimental import pallas as pl
from jax.experimental.pallas import tpu as pltpu
from jax.experimental.pallas import tpu_sc as plsc

B = 16384
F = 26
V = 100000
D = 64
NC, NS, L = 2, 16, 16
NW = NC * NS
SPW = B // NW
C = 16
NCH = SPW // C

_mesh = plsc.VectorSubcoreMesh(core_axis_name="c", subcore_axis_name="s")


@functools.partial(
    pl.kernel,
    mesh=_mesh,
    out_type=jax.ShapeDtypeStruct((B, D), jnp.float32),
    scratch_types=[
        pltpu.VMEM((F, 128), jnp.int32),
        pltpu.VMEM((F, C), jnp.int32),
        pltpu.VMEM((F, C), jnp.int32),
        pltpu.VMEM((F * C, 2 * D), jnp.float32),
        pltpu.VMEM((F * C, 2 * D), jnp.float32),
        pltpu.VMEM((C, D), jnp.float32),
        pltpu.SemaphoreType.DMA,
    ],
    compiler_params=pltpu.CompilerParams(
        use_tc_tiling_on_sc=True, needs_layout_passes=False
    ),
)
def _emb(x_hbm, tab_hbm, out_hbm, idx_v, i2_0, i2_1, r_0, r_1, out_v, sem):
    wid = lax.axis_index("s") * NC + lax.axis_index("c")
    idx2_v, rows_v = [i2_0, i2_1], [r_0, r_1]
    GC = 128 // C
    NG = NCH // GC

    def load_idx(g):
        off = pl.multiple_of(wid * SPW + g * 128, 128)
        pltpu.sync_copy(x_hbm.at[:, pl.ds(off, 128)], idx_v)

    def fire(j, b):
        for f in range(F):
            idx2_v[b][f, :] = idx_v[f, pl.ds(j * C, C)] + (f * V)
        for f in range(F):
            pltpu.async_copy(
                tab_hbm.at[idx2_v[b].at[f]], rows_v[b].at[pl.ds(f * C, C), :], sem
            )

    def drain(b):
        for f in range(F):
            pltpu.make_async_copy(
                tab_hbm.at[idx2_v[b].at[f]], rows_v[b].at[pl.ds(f * C, C), :], sem
            ).wait()

    def consume(c, b):
        @pl.loop(0, C)
        def _acc(s):
            accs = None
            for f in range(F):
                row = f * C + s
                vals = [rows_v[b][row, pl.ds(v * L, L)] for v in range(D // L)]
                accs = vals if accs is None else [a + b2 for a, b2 in zip(accs, vals)]
            for v in range(D // L):
                out_v[s, pl.ds(v * L, L)] = accs[v]

        pltpu.sync_copy(out_v, out_hbm.at[pl.ds(wid * SPW + c * C, C), :])

    load_idx(0)
    fire(0, 0)

    @pl.loop(0, NG)
    def _group(g):
        for j in range(GC):
            c = g * GC + j
            b = j % 2
            if j + 1 < GC:
                fire(j + 1, 1 - b)
                drain(b)
                consume(c, b)
            else:
                drain(b)
                consume(c, b)

                @pl.when(g + 1 < NG)
                def _():
                    load_idx(g + 1)
                    fire(0, 1 - b)


def kernel(x, tables):
    xt = x.astype(jnp.int32).T
    tabp = jnp.pad(tables, ((0, 0), (0, 0), (0, D))).reshape(F * V, 2 * D)
    return _emb(xt, tabp)
